# Initial kernel scaffold; baseline (speedup 1.0000x reference)
import jax
import jax.numpy as jnp
from jax.experimental import pallas as pl


def kernel(x, edge_index, W_l1, b_l1, W_r1, W_l2, b_l2, W_r2):
    raise NotImplementedError("write your pallas kernel here")


# speedup vs baseline: 4.7843x; 4.7843x over previous
import functools

import jax
import jax.numpy as jnp
from jax import lax
from jax.experimental import pallas as pl
from jax.experimental.pallas import tpu as pltpu
from jax.experimental.pallas import tpu_sc as plsc

N = 10000
E = 320000
D = 128
H = 128

NC = 2
NS = 16
NW = NC * NS
E_PER_W = E // NW
CHUNK = 80
NCH = E_PER_W // CHUNK
NP = 10240
ROWS_PER_TILE = NP // NS


def _fill_f32(ref, rows, val):
    v = jnp.full((16,), val, jnp.float32)
    cols = ref.shape[1] // 16

    def body(i, c):
        for j in range(cols):
            ref[i, pl.ds(j * 16, 16)] = v
        return c

    lax.fori_loop(0, rows, body, 0)


def _make_sc_agg(with_cnt: bool):
    mesh = plsc.VectorSubcoreMesh(
        core_axis_name="c", subcore_axis_name="s", num_cores=NC, num_subcores=NS
    )
    out_type = [jax.ShapeDtypeStruct((NC, NP, D), jnp.float32)]
    scratch = [
        pltpu.VMEM_SHARED((NP, D), jnp.float32),
        pltpu.VMEM((CHUNK,), jnp.int32),
        pltpu.VMEM((CHUNK,), jnp.int32),
        pltpu.VMEM((CHUNK, D), jnp.float32),
        pltpu.SemaphoreType.DMA,
    ]
    if with_cnt:
        out_type.append(jax.ShapeDtypeStruct((NC, NP, D), jnp.float32))
        scratch.append(pltpu.VMEM((CHUNK, D), jnp.float32))

    def body(x_hbm, src_hbm, dst_hbm, acc_out, *rest):
        if with_cnt:
            cnt_out, acc_sh, sidx, didx, rows, sem, ones = rest
        else:
            acc_sh, sidx, didx, rows, sem = rest
        cid = lax.axis_index("c")
        sid = lax.axis_index("s")
        wid = sid * NC + cid
        r0 = sid * ROWS_PER_TILE
        e0 = wid * E_PER_W

        def zero_own_slice():
            for z in range(ROWS_PER_TILE // CHUNK):
                pltpu.sync_copy(rows, acc_sh.at[pl.ds(r0 + z * CHUNK, CHUNK)])

        _fill_f32(rows, CHUNK, 0.0)

        if with_cnt:
            _fill_f32(ones, CHUNK, 1.0)
            zero_own_slice()
            plsc.subcore_barrier()

            def cstep(k, c):
                off = pl.multiple_of(e0 + k * CHUNK, 8)
                pltpu.sync_copy(dst_hbm.at[pl.ds(off, CHUNK)], didx)
                pltpu.sync_copy(ones, acc_sh.at[didx], add=True)
                return c

            lax.fori_loop(0, NCH, cstep, 0)
            plsc.subcore_barrier()
            pltpu.sync_copy(
                acc_sh.at[pl.ds(r0, ROWS_PER_TILE)],
                cnt_out.at[cid, pl.ds(r0, ROWS_PER_TILE)],
            )

        zero_own_slice()
        plsc.subcore_barrier()

        def step(k, c):
            off = pl.multiple_of(e0 + k * CHUNK, 8)
            pltpu.sync_copy(src_hbm.at[pl.ds(off, CHUNK)], sidx)
            pltpu.sync_copy(dst_hbm.at[pl.ds(off, CHUNK)], didx)
            pltpu.async_copy(x_hbm.at[sidx], rows, sem).wait()
            pltpu.sync_copy(rows, acc_sh.at[didx], add=True)
            return c

        lax.fori_loop(0, NCH, step, 0)
        plsc.subcore_barrier()

        pltpu.sync_copy(
            acc_sh.at[pl.ds(r0, ROWS_PER_TILE)],
            acc_out.at[cid, pl.ds(r0, ROWS_PER_TILE)],
        )

    return pl.kernel(body, out_type=tuple(out_type), mesh=mesh,
                     scratch_types=tuple(scratch))


_sc_agg_cnt = _make_sc_agg(with_cnt=True)
_sc_agg = _make_sc_agg(with_cnt=False)

R = 400
GRID = N // R


def _tc_body(last, acc_ref, cnt_ref, x_ref, wl_ref, b_ref, wr_ref, o_ref):
    agg = acc_ref[0] + acc_ref[1]
    cnt = cnt_ref[0][:, :1] + cnt_ref[1][:, :1]
    agg = agg / jnp.maximum(cnt, 1.0)
    y = (
        jnp.dot(agg, wl_ref[...], preferred_element_type=jnp.float32)
        + b_ref[...]
        + jnp.dot(x_ref[...], wr_ref[...], preferred_element_type=jnp.float32)
    )
    h = jnp.where(y > 0, y, jnp.exp(jnp.minimum(y, 0.0)) - 1.0)
    if last:
        m = jnp.max(h, axis=-1, keepdims=True)
        h = (h - m) - jnp.log(jnp.sum(jnp.exp(h - m), axis=-1, keepdims=True))
    o_ref[...] = h


def _tc_layer(acc, cnt, x, wlT, b, wrT, last):
    return pl.pallas_call(
        functools.partial(_tc_body, last),
        grid=(GRID,),
        in_specs=[
            pl.BlockSpec((NC, R, D), lambda i: (0, i, 0)),
            pl.BlockSpec((NC, R, D), lambda i: (0, i, 0)),
            pl.BlockSpec((R, D), lambda i: (i, 0)),
            pl.BlockSpec((D, H), lambda i: (0, 0)),
            pl.BlockSpec((1, H), lambda i: (0, 0)),
            pl.BlockSpec((D, H), lambda i: (0, 0)),
        ],
        out_specs=pl.BlockSpec((R, H), lambda i: (i, 0)),
        out_shape=jax.ShapeDtypeStruct((N, H), jnp.float32),
    )(acc, cnt, x, wlT, b, wrT)


def kernel(x, edge_index, W_l1, b_l1, W_r1, W_l2, b_l2, W_r2):
    src = edge_index[0]
    dst = edge_index[1]
    acc1, cnt = _sc_agg_cnt(x, src, dst)
    h = _tc_layer(acc1, cnt, x, W_l1.T, b_l1.reshape(1, H), W_r1.T, last=False)
    (acc2,) = _sc_agg(h, src, dst)
    return _tc_layer(acc2, cnt, h, W_l2.T, b_l2.reshape(1, H), W_r2.T, last=True)


# speedup vs baseline: 7.9137x; 1.6541x over previous
import functools

import jax
import jax.numpy as jnp
from jax import lax
from jax.experimental import pallas as pl
from jax.experimental.pallas import tpu as pltpu
from jax.experimental.pallas import tpu_sc as plsc

N = 10000
E = 320000
D = 128
H = 128

NC = 2
NS = 16
NW = NC * NS
E_PER_W = E // NW
CHUNK = 128
NCHM = 78
TAIL = E_PER_W - NCHM * CHUNK
NP = 10240
ROWS_PER_TILE = NP // NS


def _fill_f32(ref, rows, val):
    v = jnp.full((16,), val, jnp.float32)
    cols = ref.shape[1] // 16

    def body(i, c):
        for j in range(cols):
            ref[i, pl.ds(j * 16, 16)] = v
        return c

    lax.fori_loop(0, rows, body, 0)


def _make_sc_agg(with_cnt: bool):
    mesh = plsc.VectorSubcoreMesh(
        core_axis_name="c", subcore_axis_name="s", num_cores=NC, num_subcores=NS
    )
    out_type = [jax.ShapeDtypeStruct((NC, NP, D), jnp.float32)]
    scratch = [
        pltpu.VMEM_SHARED((NP, D), jnp.float32),
        pltpu.VMEM((2, 2, CHUNK), jnp.int32),
        pltpu.VMEM((2, TAIL), jnp.int32),
        pltpu.VMEM((CHUNK, D), jnp.float32),
        pltpu.SemaphoreType.DMA,
    ]
    if with_cnt:
        out_type.append(jax.ShapeDtypeStruct((NC, NP, D), jnp.float32))

    def body(x_hbm, em_hbm, et_hbm, acc_out, *rest):
        if with_cnt:
            cnt_out, acc_sh, ibuf, itail, rows, sem = rest
        else:
            acc_sh, ibuf, itail, rows, sem = rest
        cid = lax.axis_index("c")
        sid = lax.axis_index("s")
        wid = sid * NC + cid
        t0 = sid * ROWS_PER_TILE

        def zero_own_slice():
            for z in range(ROWS_PER_TILE // CHUNK):
                pltpu.sync_copy(rows, acc_sh.at[pl.ds(t0 + z * CHUNK, CHUNK)])

        def load_idx(k, b):
            pltpu.sync_copy(em_hbm.at[wid, k], ibuf.at[b])

        _fill_f32(rows, CHUNK, 0.0)

        if with_cnt:
            zero_own_slice()
            plsc.subcore_barrier()
            _fill_f32(rows, CHUNK, 1.0)
            load_idx(0, 0)

            def cstep(i, c):
                for b in range(2):
                    k = 2 * i + b
                    s = pltpu.async_copy(rows, acc_sh.at[ibuf.at[b, 1]],
                                         sem, add=True)
                    load_idx(k + 1, 1 - b)
                    s.wait()
                return c

            lax.fori_loop(0, NCHM // 2, cstep, 0)
            pltpu.sync_copy(et_hbm.at[wid], itail)
            pltpu.sync_copy(rows.at[pl.ds(0, TAIL)],
                            acc_sh.at[itail.at[1]], add=True)
            plsc.subcore_barrier()
            pltpu.sync_copy(
                acc_sh.at[pl.ds(t0, ROWS_PER_TILE)],
                cnt_out.at[cid, pl.ds(t0, ROWS_PER_TILE)],
            )
            _fill_f32(rows, CHUNK, 0.0)

        zero_own_slice()
        plsc.subcore_barrier()
        load_idx(0, 0)

        def step(i, c):
            for b in range(2):
                k = 2 * i + b
                g = pltpu.async_copy(x_hbm.at[ibuf.at[b, 0]], rows, sem)
                load_idx(k + 1, 1 - b)
                g.wait()
                pltpu.sync_copy(rows, acc_sh.at[ibuf.at[b, 1]], add=True)
            return c

        lax.fori_loop(0, NCHM // 2, step, 0)
        pltpu.sync_copy(et_hbm.at[wid], itail)
        pltpu.async_copy(x_hbm.at[itail.at[0]], rows.at[pl.ds(0, TAIL)],
                         sem).wait()
        pltpu.sync_copy(rows.at[pl.ds(0, TAIL)],
                        acc_sh.at[itail.at[1]], add=True)
        plsc.subcore_barrier()

        pltpu.sync_copy(
            acc_sh.at[pl.ds(t0, ROWS_PER_TILE)],
            acc_out.at[cid, pl.ds(t0, ROWS_PER_TILE)],
        )

    return pl.kernel(body, out_type=tuple(out_type), mesh=mesh,
                     scratch_types=tuple(scratch))


_sc_agg_cnt = _make_sc_agg(with_cnt=True)
_sc_agg = _make_sc_agg(with_cnt=False)

R = 400
GRID = N // R


def _tc_body(last, acc_ref, cnt_ref, x_ref, wl_ref, b_ref, wr_ref, o_ref):
    agg = acc_ref[0] + acc_ref[1]
    cnt = cnt_ref[0][:, :1] + cnt_ref[1][:, :1]
    agg = agg / jnp.maximum(cnt, 1.0)
    y = (
        jnp.dot(agg, wl_ref[...], preferred_element_type=jnp.float32)
        + b_ref[...]
        + jnp.dot(x_ref[...], wr_ref[...], preferred_element_type=jnp.float32)
    )
    h = jnp.where(y > 0, y, jnp.exp(jnp.minimum(y, 0.0)) - 1.0)
    if last:
        m = jnp.max(h, axis=-1, keepdims=True)
        h = (h - m) - jnp.log(jnp.sum(jnp.exp(h - m), axis=-1, keepdims=True))
    o_ref[...] = h


def _tc_layer(acc, cnt, x, wlT, b, wrT, last):
    return pl.pallas_call(
        functools.partial(_tc_body, last),
        grid=(GRID,),
        in_specs=[
            pl.BlockSpec((NC, R, D), lambda i: (0, i, 0)),
            pl.BlockSpec((NC, R, D), lambda i: (0, i, 0)),
            pl.BlockSpec((R, D), lambda i: (i, 0)),
            pl.BlockSpec((D, H), lambda i: (0, 0)),
            pl.BlockSpec((1, H), lambda i: (0, 0)),
            pl.BlockSpec((D, H), lambda i: (0, 0)),
        ],
        out_specs=pl.BlockSpec((R, H), lambda i: (i, 0)),
        out_shape=jax.ShapeDtypeStruct((N, H), jnp.float32),
    )(acc, cnt, x, wlT, b, wrT)


def _prep_edges(edge_index):
    per_w = edge_index.reshape(2, NW, E_PER_W)
    main = per_w[:, :, : NCHM * CHUNK].reshape(2, NW, NCHM, CHUNK)
    em = jnp.stack([main[0], main[1]], axis=2)
    em = jnp.concatenate([em, em[:, :1]], axis=1)
    tail = per_w[:, :, NCHM * CHUNK:]
    et = jnp.stack([tail[0], tail[1]], axis=1)
    return em, et


def kernel(x, edge_index, W_l1, b_l1, W_r1, W_l2, b_l2, W_r2):
    em, et = _prep_edges(edge_index)
    acc1, cnt = _sc_agg_cnt(x, em, et)
    h = _tc_layer(acc1, cnt, x, W_l1.T, b_l1.reshape(1, H), W_r1.T, last=False)
    (acc2,) = _sc_agg(h, em, et)
    return _tc_layer(acc2, cnt, h, W_l2.T, b_l2.reshape(1, H), W_r2.T, last=True)


# speedup vs baseline: 8.7493x; 1.1056x over previous
import functools

import jax
import jax.numpy as jnp
from jax import lax
from jax.experimental import pallas as pl
from jax.experimental.pallas import tpu as pltpu
from jax.experimental.pallas import tpu_sc as plsc

N = 10000
E = 320000
D = 128
H = 128

NC = 2
NS = 16
NW = NC * NS
E_PER_W = E // NW
CHUNK = 128
NCHM = 78
TAIL = E_PER_W - NCHM * CHUNK
NP = 10240
ROWS_PER_TILE = NP // NS


def _fill_f32(ref, rows, val):
    v = jnp.full((16,), val, jnp.float32)
    cols = ref.shape[1] // 16

    def body(i, c):
        for j in range(cols):
            ref[i, pl.ds(j * 16, 16)] = v
        return c

    lax.fori_loop(0, rows, body, 0)


def _make_sc_agg(with_cnt: bool):
    mesh = plsc.VectorSubcoreMesh(
        core_axis_name="c", subcore_axis_name="s", num_cores=NC, num_subcores=NS
    )
    out_type = [jax.ShapeDtypeStruct((NC, NP, D), jnp.float32)]
    scratch = [
        pltpu.VMEM_SHARED((NP, D), jnp.float32),
        pltpu.VMEM((2, 2, CHUNK), jnp.int32),
        pltpu.VMEM((2, TAIL), jnp.int32),
        pltpu.VMEM((CHUNK, D), jnp.float32),
        pltpu.VMEM((CHUNK, D), jnp.float32),
        pltpu.SemaphoreType.DMA,
        pltpu.SemaphoreType.DMA,
    ]
    if with_cnt:
        out_type.append(jax.ShapeDtypeStruct((NC, NP, D), jnp.float32))

    def body(x_hbm, em_hbm, et_hbm, acc_out, *rest):
        if with_cnt:
            cnt_out, acc_sh, ibuf, itail, rows, rowsb, gsm, ssm = rest
        else:
            acc_sh, ibuf, itail, rows, rowsb, gsm, ssm = rest
        cid = lax.axis_index("c")
        sid = lax.axis_index("s")
        wid = sid * NC + cid
        t0 = sid * ROWS_PER_TILE

        def zero_own_slice():
            for z in range(ROWS_PER_TILE // CHUNK):
                pltpu.sync_copy(rows, acc_sh.at[pl.ds(t0 + z * CHUNK, CHUNK)])

        def load_idx(k, b):
            pltpu.sync_copy(em_hbm.at[wid, k], ibuf.at[b])

        _fill_f32(rows, CHUNK, 0.0)

        if with_cnt:
            zero_own_slice()
            plsc.subcore_barrier()
            _fill_f32(rows, CHUNK, 1.0)
            load_idx(0, 0)

            def cstep(i, c):
                for b in range(2):
                    k = 2 * i + b
                    s = pltpu.async_copy(rows, acc_sh.at[ibuf.at[b, 1]],
                                         ssm, add=True)
                    load_idx(k + 1, 1 - b)
                    s.wait()
                return c

            lax.fori_loop(0, NCHM // 2, cstep, 0)
            pltpu.sync_copy(et_hbm.at[wid], itail)
            pltpu.sync_copy(rows.at[pl.ds(0, TAIL)],
                            acc_sh.at[itail.at[1]], add=True)
            plsc.subcore_barrier()
            pltpu.sync_copy(
                acc_sh.at[pl.ds(t0, ROWS_PER_TILE)],
                cnt_out.at[cid, pl.ds(t0, ROWS_PER_TILE)],
            )
            _fill_f32(rows, CHUNK, 0.0)

        zero_own_slice()
        plsc.subcore_barrier()
        load_idx(0, 0)

        def step(i, c):
            k = 2 * i
            g0 = pltpu.async_copy(x_hbm.at[ibuf.at[0, 0]], rows, gsm)
            load_idx(k + 1, 1)
            g0.wait()
            g1 = pltpu.async_copy(x_hbm.at[ibuf.at[1, 0]], rowsb, gsm)
            s0 = pltpu.async_copy(rows, acc_sh.at[ibuf.at[0, 1]], ssm,
                                  add=True)
            g1.wait()
            s0.wait()
            s1 = pltpu.async_copy(rowsb, acc_sh.at[ibuf.at[1, 1]], ssm,
                                  add=True)
            load_idx(k + 2, 0)
            s1.wait()
            return c

        lax.fori_loop(0, NCHM // 2, step, 0)
        pltpu.sync_copy(et_hbm.at[wid], itail)
        pltpu.async_copy(x_hbm.at[itail.at[0]], rows.at[pl.ds(0, TAIL)],
                         gsm).wait()
        pltpu.sync_copy(rows.at[pl.ds(0, TAIL)],
                        acc_sh.at[itail.at[1]], add=True)
        plsc.subcore_barrier()

        pltpu.sync_copy(
            acc_sh.at[pl.ds(t0, ROWS_PER_TILE)],
            acc_out.at[cid, pl.ds(t0, ROWS_PER_TILE)],
        )

    return pl.kernel(body, out_type=tuple(out_type), mesh=mesh,
                     scratch_types=tuple(scratch))


_sc_agg_cnt = _make_sc_agg(with_cnt=True)
_sc_agg = _make_sc_agg(with_cnt=False)

R = 400
GRID = N // R


def _tc_body(last, acc_ref, cnt_ref, x_ref, wl_ref, b_ref, wr_ref, o_ref):
    agg = acc_ref[0] + acc_ref[1]
    cnt = cnt_ref[0][:, :1] + cnt_ref[1][:, :1]
    agg = agg / jnp.maximum(cnt, 1.0)
    y = (
        jnp.dot(agg, wl_ref[...], preferred_element_type=jnp.float32)
        + b_ref[...]
        + jnp.dot(x_ref[...], wr_ref[...], preferred_element_type=jnp.float32)
    )
    h = jnp.where(y > 0, y, jnp.exp(jnp.minimum(y, 0.0)) - 1.0)
    if last:
        m = jnp.max(h, axis=-1, keepdims=True)
        h = (h - m) - jnp.log(jnp.sum(jnp.exp(h - m), axis=-1, keepdims=True))
    o_ref[...] = h


def _tc_layer(acc, cnt, x, wlT, b, wrT, last):
    return pl.pallas_call(
        functools.partial(_tc_body, last),
        grid=(GRID,),
        in_specs=[
            pl.BlockSpec((NC, R, D), lambda i: (0, i, 0)),
            pl.BlockSpec((NC, R, D), lambda i: (0, i, 0)),
            pl.BlockSpec((R, D), lambda i: (i, 0)),
            pl.BlockSpec((D, H), lambda i: (0, 0)),
            pl.BlockSpec((1, H), lambda i: (0, 0)),
            pl.BlockSpec((D, H), lambda i: (0, 0)),
        ],
        out_specs=pl.BlockSpec((R, H), lambda i: (i, 0)),
        out_shape=jax.ShapeDtypeStruct((N, H), jnp.float32),
    )(acc, cnt, x, wlT, b, wrT)


def _prep_edges(edge_index):
    per_w = edge_index.reshape(2, NW, E_PER_W)
    main = per_w[:, :, : NCHM * CHUNK].reshape(2, NW, NCHM, CHUNK)
    em = jnp.stack([main[0], main[1]], axis=2)
    em = jnp.concatenate([em, em[:, :1]], axis=1)
    tail = per_w[:, :, NCHM * CHUNK:]
    et = jnp.stack([tail[0], tail[1]], axis=1)
    return em, et


def kernel(x, edge_index, W_l1, b_l1, W_r1, W_l2, b_l2, W_r2):
    em, et = _prep_edges(edge_index)
    acc1, cnt = _sc_agg_cnt(x, em, et)
    h = _tc_layer(acc1, cnt, x, W_l1.T, b_l1.reshape(1, H), W_r1.T, last=False)
    (acc2,) = _sc_agg(h, em, et)
    return _tc_layer(acc2, cnt, h, W_l2.T, b_l2.reshape(1, H), W_r2.T, last=True)


# speedup vs baseline: 9.0655x; 1.0361x over previous
import functools

import jax
import jax.numpy as jnp
from jax import lax
from jax.experimental import pallas as pl
from jax.experimental.pallas import tpu as pltpu
from jax.experimental.pallas import tpu_sc as plsc

N = 10000
E = 320000
D = 128
H = 128

NC = 2
NS = 16
NW = NC * NS
E_PER_W = E // NW
CHUNK = 128
NCHM = 78
TAIL = E_PER_W - NCHM * CHUNK
NP = 10240
ROWS_PER_TILE = NP // NS


def _fill_f32(ref, rows, val):
    v = jnp.full((16,), val, jnp.float32)
    cols = ref.shape[1] // 16

    def body(i, c):
        for j in range(cols):
            ref[i, pl.ds(j * 16, 16)] = v
        return c

    lax.fori_loop(0, rows, body, 0)


def _make_sc_agg(with_cnt: bool):
    mesh = plsc.VectorSubcoreMesh(
        core_axis_name="c", subcore_axis_name="s", num_cores=NC, num_subcores=NS
    )
    out_type = [jax.ShapeDtypeStruct((NC, NP, D), jnp.float32)]
    scratch = [
        pltpu.VMEM_SHARED((NP, D), jnp.float32),
        pltpu.VMEM((3, 2, CHUNK), jnp.int32),
        pltpu.VMEM((2, TAIL), jnp.int32),
        pltpu.VMEM((CHUNK, D), jnp.float32),
        pltpu.VMEM((CHUNK, D), jnp.float32),
        pltpu.SemaphoreType.DMA,
        pltpu.SemaphoreType.DMA,
    ]
    if with_cnt:
        out_type.append(jax.ShapeDtypeStruct((NC, NP, D), jnp.float32))

    def body(x_hbm, em_hbm, et_hbm, acc_out, *rest):
        if with_cnt:
            cnt_out, acc_sh, ibuf, itail, rows, rowsb, gsm, ssm = rest
        else:
            acc_sh, ibuf, itail, rows, rowsb, gsm, ssm = rest
        cid = lax.axis_index("c")
        sid = lax.axis_index("s")
        wid = sid * NC + cid
        t0 = sid * ROWS_PER_TILE

        def zero_own_slice():
            for z in range(ROWS_PER_TILE // CHUNK):
                pltpu.sync_copy(rows, acc_sh.at[pl.ds(t0 + z * CHUNK, CHUNK)])

        def load_idx(k, b):
            pltpu.sync_copy(em_hbm.at[wid, jnp.minimum(k, NCHM - 1)],
                            ibuf.at[b])

        _fill_f32(rows, CHUNK, 0.0)

        if with_cnt:
            zero_own_slice()
            plsc.subcore_barrier()
            _fill_f32(rows, CHUNK, 1.0)
            load_idx(0, 0)

            def cstep(i, c):
                for b in range(2):
                    k = 2 * i + b
                    s = pltpu.async_copy(rows, acc_sh.at[ibuf.at[b, 1]],
                                         ssm, add=True)
                    load_idx(k + 1, 1 - b)
                    s.wait()
                return c

            lax.fori_loop(0, NCHM // 2, cstep, 0)
            pltpu.sync_copy(et_hbm.at[wid], itail)
            pltpu.sync_copy(rows.at[pl.ds(0, TAIL)],
                            acc_sh.at[itail.at[1]], add=True)
            plsc.subcore_barrier()
            pltpu.sync_copy(
                acc_sh.at[pl.ds(t0, ROWS_PER_TILE)],
                cnt_out.at[cid, pl.ds(t0, ROWS_PER_TILE)],
            )
            _fill_f32(rows, CHUNK, 0.0)

        zero_own_slice()
        plsc.subcore_barrier()
        load_idx(0, 0)

        def step(i, c):
            k = 3 * i
            g0 = pltpu.async_copy(x_hbm.at[ibuf.at[0, 0]], rows, gsm)
            load_idx(k + 1, 1)
            g0.wait()
            g1 = pltpu.async_copy(x_hbm.at[ibuf.at[1, 0]], rowsb, gsm)
            s0 = pltpu.async_copy(rows, acc_sh.at[ibuf.at[0, 1]], ssm,
                                  add=True)
            load_idx(k + 2, 2)
            g1.wait()
            s0.wait()
            g2 = pltpu.async_copy(x_hbm.at[ibuf.at[2, 0]], rows, gsm)
            s1 = pltpu.async_copy(rowsb, acc_sh.at[ibuf.at[1, 1]], ssm,
                                  add=True)
            load_idx(k + 3, 0)
            g2.wait()
            s1.wait()
            s2 = pltpu.async_copy(rows, acc_sh.at[ibuf.at[2, 1]], ssm,
                                  add=True)
            s2.wait()
            return c

        lax.fori_loop(0, NCHM // 3, step, 0)
        pltpu.sync_copy(et_hbm.at[wid], itail)
        pltpu.async_copy(x_hbm.at[itail.at[0]], rows.at[pl.ds(0, TAIL)],
                         gsm).wait()
        pltpu.sync_copy(rows.at[pl.ds(0, TAIL)],
                        acc_sh.at[itail.at[1]], add=True)
        plsc.subcore_barrier()

        pltpu.sync_copy(
            acc_sh.at[pl.ds(t0, ROWS_PER_TILE)],
            acc_out.at[cid, pl.ds(t0, ROWS_PER_TILE)],
        )

    return pl.kernel(body, out_type=tuple(out_type), mesh=mesh,
                     scratch_types=tuple(scratch))


_sc_agg_cnt = _make_sc_agg(with_cnt=True)
_sc_agg = _make_sc_agg(with_cnt=False)

R = 400
GRID = N // R


def _tc_body(last, acc_ref, cnt_ref, x_ref, wl_ref, b_ref, wr_ref, o_ref):
    agg = acc_ref[0] + acc_ref[1]
    cnt = cnt_ref[0][:, :1] + cnt_ref[1][:, :1]
    agg = agg / jnp.maximum(cnt, 1.0)
    y = (
        jnp.dot(agg, wl_ref[...], preferred_element_type=jnp.float32)
        + b_ref[...]
        + jnp.dot(x_ref[...], wr_ref[...], preferred_element_type=jnp.float32)
    )
    h = jnp.where(y > 0, y, jnp.exp(jnp.minimum(y, 0.0)) - 1.0)
    if last:
        m = jnp.max(h, axis=-1, keepdims=True)
        h = (h - m) - jnp.log(jnp.sum(jnp.exp(h - m), axis=-1, keepdims=True))
    o_ref[...] = h


def _tc_layer(acc, cnt, x, wlT, b, wrT, last):
    return pl.pallas_call(
        functools.partial(_tc_body, last),
        grid=(GRID,),
        in_specs=[
            pl.BlockSpec((NC, R, D), lambda i: (0, i, 0)),
            pl.BlockSpec((NC, R, D), lambda i: (0, i, 0)),
            pl.BlockSpec((R, D), lambda i: (i, 0)),
            pl.BlockSpec((D, H), lambda i: (0, 0)),
            pl.BlockSpec((1, H), lambda i: (0, 0)),
            pl.BlockSpec((D, H), lambda i: (0, 0)),
        ],
        out_specs=pl.BlockSpec((R, H), lambda i: (i, 0)),
        out_shape=jax.ShapeDtypeStruct((N, H), jnp.float32),
    )(acc, cnt, x, wlT, b, wrT)


def _prep_edges(edge_index):
    per_w = edge_index.reshape(2, NW, E_PER_W)
    main = per_w[:, :, : NCHM * CHUNK].reshape(2, NW, NCHM, CHUNK)
    em = jnp.stack([main[0], main[1]], axis=2)
    tail = per_w[:, :, NCHM * CHUNK:]
    et = jnp.stack([tail[0], tail[1]], axis=1)
    return em, et


def kernel(x, edge_index, W_l1, b_l1, W_r1, W_l2, b_l2, W_r2):
    em, et = _prep_edges(edge_index)
    acc1, cnt = _sc_agg_cnt(x, em, et)
    h = _tc_layer(acc1, cnt, x, W_l1.T, b_l1.reshape(1, H), W_r1.T, last=False)
    (acc2,) = _sc_agg(h, em, et)
    return _tc_layer(acc2, cnt, h, W_l2.T, b_l2.reshape(1, H), W_r2.T, last=True)


# speedup vs baseline: 9.1180x; 1.0058x over previous
import functools

import jax
import jax.numpy as jnp
from jax import lax
from jax.experimental import pallas as pl
from jax.experimental.pallas import tpu as pltpu
from jax.experimental.pallas import tpu_sc as plsc

N = 10000
E = 320000
D = 128
H = 128

NC = 2
NS = 16
NW = NC * NS
E_PER_W = E // NW
CHUNK = 128
NCHM = 78
TAIL = E_PER_W - NCHM * CHUNK
NP = 10240
ROWS_PER_TILE = NP // NS


def _fill_f32(ref, rows, val):
    v = jnp.full((16,), val, jnp.float32)
    cols = ref.shape[1] // 16

    def body(i, c):
        for j in range(cols):
            ref[i, pl.ds(j * 16, 16)] = v
        return c

    lax.fori_loop(0, rows, body, 0)


def _make_sc_agg(with_cnt: bool):
    mesh = plsc.VectorSubcoreMesh(
        core_axis_name="c", subcore_axis_name="s", num_cores=NC, num_subcores=NS
    )
    out_type = [jax.ShapeDtypeStruct((NC, NP, D), jnp.float32)]
    scratch = [
        pltpu.VMEM_SHARED((NP, D), jnp.float32),
        pltpu.VMEM((3, 2, CHUNK), jnp.int32),
        pltpu.VMEM((2, TAIL), jnp.int32),
        pltpu.VMEM((CHUNK, D), jnp.float32),
        pltpu.VMEM((CHUNK, D), jnp.float32),
        pltpu.SemaphoreType.DMA,
        pltpu.SemaphoreType.DMA,
    ]
    if with_cnt:
        out_type.append(jax.ShapeDtypeStruct((NC, NP, D), jnp.float32))

    def body(x_hbm, em_hbm, et_hbm, acc_out, *rest):
        if with_cnt:
            cnt_out, acc_sh, ibuf, itail, rows, rowsb, gsm, ssm = rest
        else:
            acc_sh, ibuf, itail, rows, rowsb, gsm, ssm = rest
        cid = lax.axis_index("c")
        sid = lax.axis_index("s")
        wid = sid * NC + cid
        t0 = sid * ROWS_PER_TILE

        def zero_own_slice():
            for z in range(ROWS_PER_TILE // CHUNK):
                pltpu.sync_copy(rows, acc_sh.at[pl.ds(t0 + z * CHUNK, CHUNK)])

        def load_idx(k, b):
            pltpu.sync_copy(em_hbm.at[wid, jnp.minimum(k, NCHM - 1)],
                            ibuf.at[b])

        _fill_f32(rows, CHUNK, 0.0)

        if with_cnt:
            zero_own_slice()
            plsc.subcore_barrier()
            _fill_f32(rows, CHUNK, 1.0)
            load_idx(0, 0)

            def cstep(i, c):
                for b in range(2):
                    k = 2 * i + b
                    s = pltpu.async_copy(rows, acc_sh.at[ibuf.at[b, 1]],
                                         ssm, add=True)
                    load_idx(k + 1, 1 - b)
                    s.wait()
                return c

            lax.fori_loop(0, NCHM // 2, cstep, 0)
            pltpu.sync_copy(et_hbm.at[wid], itail)
            pltpu.sync_copy(rows.at[pl.ds(0, TAIL)],
                            acc_sh.at[itail.at[1]], add=True)
            plsc.subcore_barrier()
            pltpu.sync_copy(
                acc_sh.at[pl.ds(t0, ROWS_PER_TILE)],
                cnt_out.at[cid, pl.ds(t0, ROWS_PER_TILE)],
            )
            _fill_f32(rows, CHUNK, 0.0)

        zero_own_slice()
        plsc.subcore_barrier()
        load_idx(0, 0)

        def step(i, c):
            k = 3 * i
            g0 = pltpu.async_copy(x_hbm.at[ibuf.at[0, 0]], rows, gsm)
            load_idx(k + 1, 1)
            g0.wait()
            g1 = pltpu.async_copy(x_hbm.at[ibuf.at[1, 0]], rowsb, gsm)
            s0 = pltpu.async_copy(rows, acc_sh.at[ibuf.at[0, 1]], ssm,
                                  add=True)
            load_idx(k + 2, 2)
            g1.wait()
            s0.wait()
            g2 = pltpu.async_copy(x_hbm.at[ibuf.at[2, 0]], rows, gsm)
            s1 = pltpu.async_copy(rowsb, acc_sh.at[ibuf.at[1, 1]], ssm,
                                  add=True)
            load_idx(k + 3, 0)
            g2.wait()
            s1.wait()
            s2 = pltpu.async_copy(rows, acc_sh.at[ibuf.at[2, 1]], ssm,
                                  add=True)
            s2.wait()
            return c

        lax.fori_loop(0, NCHM // 3, step, 0)
        pltpu.sync_copy(et_hbm.at[wid], itail)
        pltpu.async_copy(x_hbm.at[itail.at[0]], rows.at[pl.ds(0, TAIL)],
                         gsm).wait()
        pltpu.sync_copy(rows.at[pl.ds(0, TAIL)],
                        acc_sh.at[itail.at[1]], add=True)
        plsc.subcore_barrier()

        pltpu.sync_copy(
            acc_sh.at[pl.ds(t0, ROWS_PER_TILE)],
            acc_out.at[cid, pl.ds(t0, ROWS_PER_TILE)],
        )

    return pl.kernel(body, out_type=tuple(out_type), mesh=mesh,
                     scratch_types=tuple(scratch))


_sc_agg_cnt = _make_sc_agg(with_cnt=True)
_sc_agg = _make_sc_agg(with_cnt=False)

R = 400
GRID = N // R


def _tc_body(last, acc_ref, cnt_ref, x_ref, wl_ref, b_ref, wr_ref, o_ref):
    agg = acc_ref[0] + acc_ref[1]
    cnt = cnt_ref[0][:, :1] + cnt_ref[1][:, :1]
    agg = agg / jnp.maximum(cnt, 1.0)
    dn = (((1,), (1,)), ((), ()))
    y = (
        lax.dot_general(agg, wl_ref[...], dn,
                        preferred_element_type=jnp.float32)
        + b_ref[...]
        + lax.dot_general(x_ref[...], wr_ref[...], dn,
                          preferred_element_type=jnp.float32)
    )
    h = jnp.where(y > 0, y, jnp.exp(jnp.minimum(y, 0.0)) - 1.0)
    if last:
        m = jnp.max(h, axis=-1, keepdims=True)
        h = (h - m) - jnp.log(jnp.sum(jnp.exp(h - m), axis=-1, keepdims=True))
    o_ref[...] = h


def _tc_layer(acc, cnt, x, wlT, b, wrT, last):
    return pl.pallas_call(
        functools.partial(_tc_body, last),
        grid=(GRID,),
        in_specs=[
            pl.BlockSpec((NC, R, D), lambda i: (0, i, 0)),
            pl.BlockSpec((NC, R, D), lambda i: (0, i, 0)),
            pl.BlockSpec((R, D), lambda i: (i, 0)),
            pl.BlockSpec((D, H), lambda i: (0, 0)),
            pl.BlockSpec((1, H), lambda i: (0, 0)),
            pl.BlockSpec((D, H), lambda i: (0, 0)),
        ],
        out_specs=pl.BlockSpec((R, H), lambda i: (i, 0)),
        out_shape=jax.ShapeDtypeStruct((N, H), jnp.float32),
    )(acc, cnt, x, wlT, b, wrT)


def _prep_edges(edge_index):
    per_w = edge_index.reshape(2, NW, E_PER_W)
    main = per_w[:, :, : NCHM * CHUNK].reshape(2, NW, NCHM, CHUNK)
    em = jnp.stack([main[0], main[1]], axis=2)
    tail = per_w[:, :, NCHM * CHUNK:]
    et = jnp.stack([tail[0], tail[1]], axis=1)
    return em, et


def kernel(x, edge_index, W_l1, b_l1, W_r1, W_l2, b_l2, W_r2):
    em, et = _prep_edges(edge_index)
    acc1, cnt = _sc_agg_cnt(x, em, et)
    h = _tc_layer(acc1, cnt, x, W_l1, b_l1.reshape(1, H), W_r1, last=False)
    (acc2,) = _sc_agg(h, em, et)
    return _tc_layer(acc2, cnt, h, W_l2, b_l2.reshape(1, H), W_r2, last=True)
